# baseline (speedup 1.0000x reference)
import jax
import jax.numpy as jnp
from jax.experimental import pallas as pl
from jax.experimental.pallas import tpu as pltpu

_TB = 256


def _body(msg_ref, ent_any, out_ref, ment_ref, mnn_ref, ent_v, sem):
    out_ref[...] = msg_ref[...]

    @pl.when(pl.program_id(0) == 0)
    def _():
        cp = pltpu.make_async_copy(ent_any, ent_v, sem)
        cp.start()
        cp.wait()
        s = jnp.sum(ent_v[...], axis=1)
        ment_ref[...] = s
        mnn_ref[...] = s


def kernel(messages, apply_noise, entropy):
    B, L, V = messages.shape
    out, ment, mnn = pl.pallas_call(
        _body,
        grid=(B // _TB,),
        in_specs=[
            pl.BlockSpec((_TB, L, V), lambda i: (i, 0, 0)),
            pl.BlockSpec(memory_space=pl.ANY),
        ],
        out_specs=[
            pl.BlockSpec((_TB, L, V), lambda i: (i, 0, 0)),
            pl.BlockSpec((B,), lambda i: (0,)),
            pl.BlockSpec((B,), lambda i: (0,)),
        ],
        out_shape=[
            jax.ShapeDtypeStruct((B, L, V), messages.dtype),
            jax.ShapeDtypeStruct((B,), entropy.dtype),
            jax.ShapeDtypeStruct((B,), entropy.dtype),
        ],
        scratch_shapes=[
            pltpu.VMEM((B, L), jnp.float32),
            pltpu.SemaphoreType.DMA,
        ],
    )(messages, entropy)
    sent = entropy + jnp.zeros((), entropy.dtype)
    snn = entropy + jnp.zeros((), entropy.dtype)
    return (out, ment, sent, mnn, snn)


# speedup vs baseline: 1.0355x; 1.0355x over previous
import jax
import jax.numpy as jnp
from jax.experimental import pallas as pl

_TB = 256


def _body(msg_ref, ent_ref, out_ref, ment_ref, mnn_ref):
    out_ref[...] = msg_ref[...]
    s = jnp.sum(ent_ref[...], axis=1)
    ment_ref[...] = s
    mnn_ref[...] = s


def kernel(messages, apply_noise, entropy):
    B, L, V = messages.shape
    out, ment, mnn = pl.pallas_call(
        _body,
        grid=(B // _TB,),
        in_specs=[
            pl.BlockSpec((_TB, L, V), lambda i: (i, 0, 0)),
            pl.BlockSpec((_TB, L), lambda i: (i, 0)),
        ],
        out_specs=[
            pl.BlockSpec((_TB, L, V), lambda i: (i, 0, 0)),
            pl.BlockSpec((_TB,), lambda i: (i,)),
            pl.BlockSpec((_TB,), lambda i: (i,)),
        ],
        out_shape=[
            jax.ShapeDtypeStruct((B, L, V), messages.dtype),
            jax.ShapeDtypeStruct((B,), entropy.dtype),
            jax.ShapeDtypeStruct((B,), entropy.dtype),
        ],
    )(messages, entropy)
    sent = entropy + jnp.zeros((), entropy.dtype)
    snn = entropy + jnp.zeros((), entropy.dtype)
    return (out, ment, sent, mnn, snn)
